# baseline (speedup 1.0000x reference)
import jax
import jax.numpy as jnp
from jax.experimental import pallas as pl
from jax.experimental.pallas import tpu as pltpu

_INV_SQRT_2PI = 0.3989422804014327


def _bf(v):
    return v.astype(jnp.bfloat16).astype(jnp.float32)


def _row(w_ref, i):
    return jnp.full((1, 128), w_ref[i], dtype=jnp.float32)


def _cppn_body(w1_ref, w2_ref, wo_ref, x_ref, y_ref, z_ref, out_ref, wz_ref):
    @pl.when(pl.program_id(0) == 0)
    def _init():
        wz_ref[0] = _bf(jnp.concatenate([_row(w1_ref, 2 + s) for s in range(8)], 0))
        wz_ref[1] = _bf(jnp.concatenate([_row(w1_ref, 10 + s) for s in range(8)], 0))
        wz_ref[2] = _bf(jnp.concatenate([_row(w2_ref, 3 + s) for s in range(8)], 0))
        wz_ref[3] = _bf(jnp.concatenate([_row(w2_ref, 11 + s) for s in range(8)], 0))

    x = x_ref[...]
    y = y_ref[...]
    zb0 = _bf(z_ref[0])
    zb1 = _bf(z_ref[1])
    s1 = (w1_ref[0] * x + w1_ref[1] * y
          + jnp.sum(zb0 * wz_ref[0] + zb1 * wz_ref[1], axis=1))
    s2 = (w2_ref[0] * x + w2_ref[1] * y
          + jnp.sum(zb0 * wz_ref[2] + zb1 * wz_ref[3], axis=1))
    h1 = jnp.sin(s1)
    pre2 = s2 + w2_ref[2] * h1
    h2 = _INV_SQRT_2PI * jnp.exp(-0.5 * pre2 * pre2)
    h1b = _bf(h1)
    h2b = _bf(h2)
    o = []
    for j in range(3):
        p = _bf(wo_ref[0, j]) * h1b + _bf(wo_ref[1, j]) * h2b
        o.append(1.0 / (1.0 + jnp.exp(-p)))
    o.append(o[2])
    out_ref[...] = jnp.stack(o, axis=1).reshape(out_ref.shape)


@jax.jit
def _run(X, Y, Z, w1, w2, w_out):
    N = X.shape[0]
    M = N // 128
    BM = 256

    Xr = X.reshape(M, 128)
    Yr = Y.reshape(M, 128)
    Zr = Z.reshape(M, 128, 2, 8).transpose(2, 0, 3, 1)

    out4 = pl.pallas_call(
        _cppn_body,
        grid=(M // BM,),
        in_specs=[
            pl.BlockSpec(memory_space=pltpu.SMEM),
            pl.BlockSpec(memory_space=pltpu.SMEM),
            pl.BlockSpec(memory_space=pltpu.SMEM),
            pl.BlockSpec((BM, 128), lambda i: (i, 0)),
            pl.BlockSpec((BM, 128), lambda i: (i, 0)),
            pl.BlockSpec((2, BM, 8, 128), lambda i: (0, i, 0, 0)),
        ],
        out_specs=pl.BlockSpec((4 * BM, 128), lambda i: (i, 0)),
        out_shape=jax.ShapeDtypeStruct((4 * M, 128), jnp.float32),
        scratch_shapes=[pltpu.VMEM((4, 8, 128), jnp.float32)],
    )(w1, w2, w_out, Xr, Yr, Zr)
    return out4.reshape(M, 4, 128).transpose(0, 2, 1).reshape(N, 4)[:, :3]


def kernel(X, Y, R, Z, w1, w2, w_out):
    del R
    return _run(X, Y, Z, w1, w2, w_out)
